# Initial kernel scaffold; baseline (speedup 1.0000x reference)
import jax
import jax.numpy as jnp
from jax.experimental import pallas as pl


def kernel(x):
    raise NotImplementedError("write your pallas kernel here")


# speedup vs baseline: 134.6902x; 134.6902x over previous
import functools

import jax
import jax.numpy as jnp
from jax.experimental import pallas as pl


def _hist_body(x_ref, o_ref):
    g8 = x_ref[0, 8]
    g9 = x_ref[0, 9]
    mag = jnp.sqrt(g8 * g8 + g9 * g9)
    best = x_ref[0, 0]
    bi = jnp.zeros_like(best, dtype=jnp.int32)
    for c in range(1, 8):
        v = x_ref[0, c]
        gt = v > best
        best = jnp.where(gt, v, best)
        bi = jnp.where(gt, c, bi)
    for c in range(8):
        o_ref[0, c] = jnp.where(bi == c, mag, 0.0)


def kernel(x):
    B, C, H, W = x.shape
    R = 64
    grid = (B, H // R)
    out = pl.pallas_call(
        _hist_body,
        grid=grid,
        in_specs=[pl.BlockSpec((1, C, R, W), lambda b, r: (b, 0, r, 0))],
        out_specs=pl.BlockSpec((1, 8, R, W), lambda b, r: (b, 0, r, 0)),
        out_shape=jax.ShapeDtypeStruct((B, 8, H, W), x.dtype),
    )(x)
    return out
